# Initial kernel scaffold; baseline (speedup 1.0000x reference)
import jax
import jax.numpy as jnp
from jax.experimental import pallas as pl


def kernel(input, W_tok, W_pos):
    raise NotImplementedError("write your pallas kernel here")


# speedup vs baseline: 6.2859x; 6.2859x over previous
import functools

import jax
import jax.numpy as jnp
from jax import lax
from jax.experimental import pallas as pl
from jax.experimental.pallas import tpu as pltpu
from jax.experimental.pallas import tpu_sc as plsc

_VOCAB = 100000
_EMBED = 64
_SEQ = 200

_NC = 2
_NS = 16
_NW = _NC * _NS
_LANES = 16

_IDX_MINOR = 100
_CHUNK_ROWS = 400
_CR = _CHUNK_ROWS // _IDX_MINOR


def _sc_body(n_rows, wt_hbm, wp_hbm, idx_hbm, out_hbm,
             pos_v, idx_v, rows_v, sem):
    per_w = n_rows // _NW
    n_chunks = per_w // _CHUNK_ROWS
    idx_rows_per_w = per_w // _IDX_MINOR

    wid = lax.axis_index("s") * _NC + lax.axis_index("c")

    pltpu.sync_copy(wp_hbm.at[pl.ds(0, _SEQ)], pos_v)

    def chunk_body(c, carry):
        row_base = wid * per_w + c * _CHUNK_ROWS
        irow = wid * idx_rows_per_w + c * _CR

        pltpu.sync_copy(idx_hbm.at[pl.ds(irow, _CR)], idx_v)

        copies = []
        for i in range(_CR):
            copies.append(pltpu.async_copy(
                wt_hbm.at[idx_v.at[i]],
                rows_v.at[pl.ds(i * _IDX_MINOR, _IDX_MINOR)],
                sem))
        for d in copies:
            d.wait()

        def add_body(j, carry2):
            pos_regs = [pos_v[j, pl.ds(l * _LANES, _LANES)]
                        for l in range(_EMBED // _LANES)]
            for rep in range(_CHUNK_ROWS // _SEQ):
                r = rep * _SEQ + j
                for l in range(_EMBED // _LANES):
                    sl = pl.ds(l * _LANES, _LANES)
                    rows_v[r, sl] = rows_v[r, sl] + pos_regs[l]
            return carry2

        lax.fori_loop(0, _SEQ, add_body, 0)

        pltpu.sync_copy(rows_v, out_hbm.at[pl.ds(row_base, _CHUNK_ROWS)])
        return carry

    lax.fori_loop(0, n_chunks, chunk_body, 0)


def kernel(input, W_tok, W_pos):
    batch, seq = input.shape
    n_rows = batch * seq
    ids_flat = input.reshape(n_rows // _IDX_MINOR, _IDX_MINOR).astype(jnp.int32)

    mesh = plsc.VectorSubcoreMesh(core_axis_name="c", subcore_axis_name="s",
                                  num_cores=_NC, num_subcores=_NS)
    out = pl.kernel(
        functools.partial(_sc_body, n_rows),
        out_type=jax.ShapeDtypeStruct((n_rows, _EMBED), jnp.float32),
        mesh=mesh,
        scratch_types=[
            pltpu.VMEM((_SEQ, _EMBED), jnp.float32),
            pltpu.VMEM((_CR, _IDX_MINOR), jnp.int32),
            pltpu.VMEM((_CHUNK_ROWS, _EMBED), jnp.float32),
            pltpu.SemaphoreType.DMA,
        ],
        compiler_params=pltpu.CompilerParams(use_tc_tiling_on_sc=False),
    )(W_tok, W_pos, ids_flat)
    return out.reshape(batch, seq, _EMBED)
